# baseline (speedup 1.0000x reference)
import functools

import jax
import jax.numpy as jnp
from jax.experimental import pallas as pl
from jax.experimental.pallas import tpu as pltpu
from jax.experimental.pallas import tpu_sc as plsc

_NUM_E = 8192
_DIM = 64
_N = 16384
_BLK = 256
_NBLK = _N // _BLK
_COMMIT = 0.25
_LANES = 128
_CCH = 4096


def _chunk_tournament(xn, en, mm2c, col0):
    rows = xn.shape[0]
    iota = jax.lax.broadcasted_iota(jnp.int32, (rows, _LANES), 1)
    big = jnp.int32(1 << 30)
    acc_v = None
    for b in range(_CCH // _LANES):
        off = b * _LANES
        t = xn + en[:, col0 + off:col0 + off + _LANES]
        cand_v = t + mm2c[:, off:off + _LANES]
        if acc_v is None:
            acc_v, acc_i = cand_v, iota + (col0 + off)
        else:
            win = cand_v < acc_v
            acc_v = jnp.where(win, cand_v, acc_v)
            acc_i = jnp.where(win, iota + (col0 + off), acc_i)
    row_v = jnp.min(acc_v, axis=1)
    tied = acc_v == row_v[:, None]
    row_i = jnp.min(jnp.where(tied, acc_i, big), axis=1)
    return row_v, row_i


def _dist_body(x_ref, xn_ref, et_ref, en_ref, idx_ref, loss_ref):
    x = x_ref[...]
    xn = xn_ref[...]
    x2b = (-2.0 * x).astype(jnp.bfloat16)
    en = en_ref[...]
    half = _NUM_E // 2
    hv = [None, None]
    hi = [None, None]
    for c in range(0, _NUM_E, _CCH):
        mm2c = jax.lax.dot_general(
            x2b, et_ref[:, c:c + _CCH], (((1,), (0,)), ((), ())),
            preferred_element_type=jnp.float32)
        cv, ci = _chunk_tournament(xn, en, mm2c, c)
        h = c // half
        if hv[h] is None:
            hv[h], hi[h] = cv, ci
        else:
            win = cv < hv[h]
            hv[h] = jnp.where(win, cv, hv[h])
            hi[h] = jnp.where(win, ci, hi[h])
    v1, i1, v2, i2 = hv[0], hi[0], hv[1], hi[1]
    v1r = v1.astype(jnp.bfloat16).astype(jnp.float32)
    win2 = v2 < v1r
    idx_ref[0, 0, :] = jnp.where(win2, i2, i1)
    m = jnp.where(win2, v2, v1)

    @pl.when(pl.program_id(0) == 0)
    def _init():
        loss_ref[...] = jnp.zeros((1, 1), jnp.float32)

    loss_ref[...] += jnp.sum(m).reshape(1, 1)


def _distances_argmin(x_flat, xn, et, en):
    return pl.pallas_call(
        _dist_body,
        grid=(_NBLK,),
        in_specs=[
            pl.BlockSpec((_BLK, _DIM), lambda i: (i, 0)),
            pl.BlockSpec((_BLK, 1), lambda i: (i, 0)),
            pl.BlockSpec((_DIM, _NUM_E), lambda i: (0, 0)),
            pl.BlockSpec((1, _NUM_E), lambda i: (0, 0)),
        ],
        out_specs=[
            pl.BlockSpec((1, 1, _BLK), lambda i: (i, 0, 0)),
            pl.BlockSpec((1, 1), lambda i: (0, 0)),
        ],
        out_shape=[
            jax.ShapeDtypeStruct((_NBLK, 1, _BLK), jnp.int32),
            jax.ShapeDtypeStruct((1, 1), jnp.float32),
        ],
    )(x_flat, xn, et, en)


_SC_CORES = 2
_SC_SUBCORES = 16
_SC_WORKERS = _SC_CORES * _SC_SUBCORES
_BPW = _N // _SC_WORKERS
_CHUNK = 128
_NCHUNK = _BPW // _CHUNK
_TW = 128


def _sc_gather(emb, idx_flat):
    mesh = plsc.VectorSubcoreMesh(core_axis_name="c", subcore_axis_name="s")

    @functools.partial(
        pl.kernel, mesh=mesh,
        out_type=jax.ShapeDtypeStruct((_N, _TW), emb.dtype),
        scratch_types=[
            pltpu.VMEM((_CHUNK,), jnp.int32),
            pltpu.VMEM((_CHUNK, _TW), jnp.float32),
            pltpu.SemaphoreType.DMA,
        ],
    )
    def k(table_hbm, idx_hbm, out_hbm, idx_v, rows_v, sem):
        wid = jax.lax.axis_index("s") * _SC_CORES + jax.lax.axis_index("c")
        base = wid * _BPW

        @pl.loop(0, _NCHUNK)
        def _(c):
            off = base + c * _CHUNK
            pltpu.sync_copy(idx_hbm.at[pl.ds(off, _CHUNK)], idx_v)
            pltpu.async_copy(table_hbm.at[idx_v], rows_v, sem).wait()
            pltpu.sync_copy(rows_v, out_hbm.at[pl.ds(off, _CHUNK)])

    return k(emb, idx_flat)


def kernel(inputs, embedding):
    xc = jnp.clip(inputs, -1.0, 1.0)
    xn = jnp.sum(xc ** 2, axis=2).reshape(_N, 1)
    x_flat = xc.reshape(_N, _DIM)
    et = embedding.T.astype(jnp.bfloat16)
    en = jnp.sum(embedding ** 2, axis=1)[None, :]
    idx3, loss_acc = _distances_argmin(x_flat, xn, et, en)
    idx_flat = idx3.reshape(_N)
    table = jnp.pad(embedding, ((0, 0), (0, _TW - _DIM)))
    gathered = _sc_gather(table, idx_flat)
    quantized = gathered[:, :_DIM].reshape(inputs.shape)
    loss = (1.0 + _COMMIT) * loss_acc[0, 0] / jnp.float32(_N * _DIM)
    return (quantized, loss)
